# baseline (speedup 1.0000x reference)
import math

import jax
import jax.numpy as jnp
from jax.experimental import pallas as pl
from jax.experimental.pallas import tpu as pltpu

_DIM = 2048
_H = 16
_DH = 128
_S = 2048
_KEEP = max(1, int(_S * (1.0 - 0.6)))
_BLK = 256
_NB = _S // _BLK
_SCALE = 1.0 / math.sqrt(_DH)
_NEG = -1e9


def _key_to_f32(cand_u):
    return jax.lax.bitcast_convert_type(
        jnp.where(cand_u >= jnp.uint32(0x80000000),
                  cand_u & jnp.uint32(0x7FFFFFFF), ~cand_u),
        jnp.float32)


def _mono_kernel(x_ref, wq_ref, bq_ref, wk_ref, bk_ref, wv_ref, bv_ref,
                 wo_ref, bo_ref, o_ref, k_s, v_s, o_s):
    xh = x_ref[...]
    k_s[...] = jnp.dot(xh, wk_ref[0], preferred_element_type=jnp.float32) + bk_ref[0]
    v_s[...] = jnp.dot(xh, wv_ref[0], preferred_element_type=jnp.float32) + bv_ref[0]

    for i in range(_NB):
        ncol = (i + 1) * _BLK
        q = jnp.dot(xh[i * _BLK:(i + 1) * _BLK, :], wq_ref[0],
                    preferred_element_type=jnp.float32) + bq_ref[0]
        scores = jax.lax.dot_general(
            q, k_s[0:ncol, :], (((1,), (1,)), ((), ())),
            preferred_element_type=jnp.float32) * _SCALE
        rows = i * _BLK + jax.lax.broadcasted_iota(jnp.int32, (_BLK, ncol), 0)
        cols = jax.lax.broadcasted_iota(jnp.int32, (_BLK, ncol), 1)
        scores = jnp.where(cols <= rows, scores, _NEG)
        m = jnp.max(scores, axis=1, keepdims=True)
        p = jnp.exp(scores - m)
        if ncol >= _KEEP:
            u = jax.lax.bitcast_convert_type(scores, jnp.uint32)
            chopped = jax.lax.bitcast_convert_type(
                (u >> 16).astype(jnp.uint16), jnp.bfloat16)
            nchunk = ncol // _BLK
            prefix = jnp.zeros((_BLK, 1), jnp.uint32)
            for bit in range(15, -1, -1):
                c = prefix | jnp.uint32(1 << bit)
                b16 = jnp.where(c >= jnp.uint32(0x8000),
                                c & jnp.uint32(0x7FFF), ~c & jnp.uint32(0xFFFF))
                fc16 = jax.lax.bitcast_convert_type(
                    b16.astype(jnp.uint16), jnp.bfloat16)
                ind = (chopped >= fc16).astype(jnp.bfloat16)
                acc = ind[:, 0:_BLK]
                for j in range(1, nchunk):
                    acc = acc + ind[:, j * _BLK:(j + 1) * _BLK]
                cnt = jnp.sum(acc.astype(jnp.float32), axis=1, keepdims=True)
                prefix = jnp.where(cnt >= float(_KEEP), c, prefix)
            prefix = prefix << 16
            for bit in range(15, -1, -1):
                fc = _key_to_f32(prefix | jnp.uint32(1 << bit))
                cnt = jnp.sum((scores >= fc).astype(jnp.float32),
                              axis=1, keepdims=True)
                prefix = jnp.where(cnt >= float(_KEEP),
                                   prefix | jnp.uint32(1 << bit), prefix)
            p = jnp.where(scores >= _key_to_f32(prefix), p, 0.0)
        denom = jnp.sum(p, axis=1, keepdims=True)
        o_s[i * _BLK:(i + 1) * _BLK, :] = jnp.dot(
            p, v_s[0:ncol, :], preferred_element_type=jnp.float32) / denom

    o_ref[...] = jax.lax.dot_general(
        o_s[...], wo_ref[...], (((0,), (1,)), ((), ())),
        preferred_element_type=jnp.float32) + bo_ref[0]


def kernel(x, causal_mask, Wq, bq, Wk, bk, Wv, bv, Wo, bo):
    x2 = x.reshape(_S, _DIM)
    w_spec = pl.BlockSpec((1, _DH, _DH), lambda h: (h, 0, 0))
    b_spec = pl.BlockSpec((1, 1, _DH), lambda h: (h, 0, 0))
    final = pl.pallas_call(
        _mono_kernel,
        grid=(_H,),
        in_specs=[
            pl.BlockSpec((_S, _DH), lambda h: (0, h)),
            w_spec, b_spec, w_spec, b_spec, w_spec, b_spec,
            pl.BlockSpec((_DIM, _DIM), lambda h: (0, 0)),
            pl.BlockSpec((1, _DIM), lambda h: (0, 0)),
        ],
        out_specs=pl.BlockSpec((_DH, _DIM), lambda h: (h, 0)),
        out_shape=jax.ShapeDtypeStruct((_S, _DIM), jnp.float32),
        scratch_shapes=[
            pltpu.VMEM((_S, _DH), jnp.float32),
            pltpu.VMEM((_S, _DH), jnp.float32),
            pltpu.VMEM((_S, _DH), jnp.float32),
        ],
    )(x2, Wq, bq.reshape(_H, 1, _DH), Wk, bk.reshape(_H, 1, _DH),
      Wv, bv.reshape(_H, 1, _DH), Wo, bo.reshape(1, _DIM))
    return final.reshape(1, _S, _DIM)


# speedup vs baseline: 1.4077x; 1.4077x over previous
import math

import jax
import jax.numpy as jnp
from jax.experimental import pallas as pl
from jax.experimental.pallas import tpu as pltpu

_DIM = 2048
_H = 16
_DH = 128
_S = 2048
_KEEP = max(1, int(_S * (1.0 - 0.6)))
_BLK = 256
_NB = _S // _BLK
_SCALE = 1.0 / math.sqrt(_DH)
_NEG = -1e9


def _key_to_f32(cand_u):
    return jax.lax.bitcast_convert_type(
        jnp.where(cand_u >= jnp.uint32(0x80000000),
                  cand_u & jnp.uint32(0x7FFFFFFF), ~cand_u),
        jnp.float32)


def _mono_kernel(x_ref, wq_ref, bq_ref, wk_ref, bk_ref, wv_ref, bv_ref,
                 wo_ref, bo_ref, o_ref, k_s, v_s, o_s):
    xh = x_ref[...]
    k_s[...] = jnp.dot(xh, wk_ref[0], preferred_element_type=jnp.float32) + bk_ref[0]
    v_s[...] = jnp.dot(xh, wv_ref[0], preferred_element_type=jnp.float32) + bv_ref[0]

    for i in range(_NB):
        ncol = (i + 1) * _BLK
        q = jnp.dot(xh[i * _BLK:(i + 1) * _BLK, :], wq_ref[0],
                    preferred_element_type=jnp.float32) + bq_ref[0]
        scores = jax.lax.dot_general(
            q, k_s[0:ncol, :], (((1,), (1,)), ((), ())),
            preferred_element_type=jnp.float32) * _SCALE
        rows = i * _BLK + jax.lax.broadcasted_iota(jnp.int32, (_BLK, ncol), 0)
        cols = jax.lax.broadcasted_iota(jnp.int32, (_BLK, ncol), 1)
        scores = jnp.where(cols <= rows, scores, _NEG)
        m = jnp.max(scores, axis=1, keepdims=True)
        if ncol >= _KEEP:
            prefix = jnp.zeros((_BLK, 1), jnp.uint32)
            for bit in range(31, -1, -1):
                fc = _key_to_f32(prefix | jnp.uint32(1 << bit))
                cnt = jnp.sum((scores >= fc).astype(jnp.float32),
                              axis=1, keepdims=True)
                prefix = jnp.where(cnt >= float(_KEEP),
                                   prefix | jnp.uint32(1 << bit), prefix)
            p = jnp.where(scores >= _key_to_f32(prefix),
                          jnp.exp(scores - m), 0.0)
        else:
            p = jnp.exp(scores - m)
        denom = jnp.sum(p, axis=1, keepdims=True)
        o_s[i * _BLK:(i + 1) * _BLK, :] = jnp.dot(
            p, v_s[0:ncol, :], preferred_element_type=jnp.float32) / denom

    o_ref[...] = jax.lax.dot_general(
        o_s[...], wo_ref[...], (((0,), (1,)), ((), ())),
        preferred_element_type=jnp.float32) + bo_ref[0]


def kernel(x, causal_mask, Wq, bq, Wk, bk, Wv, bv, Wo, bo):
    x2 = x.reshape(_S, _DIM)
    w_spec = pl.BlockSpec((1, _DH, _DH), lambda h: (h, 0, 0))
    b_spec = pl.BlockSpec((1, 1, _DH), lambda h: (h, 0, 0))
    final = pl.pallas_call(
        _mono_kernel,
        grid=(_H,),
        in_specs=[
            pl.BlockSpec((_S, _DH), lambda h: (0, h)),
            w_spec, b_spec, w_spec, b_spec, w_spec, b_spec,
            pl.BlockSpec((_DIM, _DIM), lambda h: (0, 0)),
            pl.BlockSpec((1, _DIM), lambda h: (0, 0)),
        ],
        out_specs=pl.BlockSpec((_DH, _DIM), lambda h: (h, 0)),
        out_shape=jax.ShapeDtypeStruct((_S, _DIM), jnp.float32),
        scratch_shapes=[
            pltpu.VMEM((_S, _DH), jnp.float32),
            pltpu.VMEM((_S, _DH), jnp.float32),
            pltpu.VMEM((_S, _DH), jnp.float32),
        ],
    )(x2, Wq, bq.reshape(_H, 1, _DH), Wk, bk.reshape(_H, 1, _DH),
      Wv, bv.reshape(_H, 1, _DH), Wo, bo.reshape(1, _DIM))
    return final.reshape(1, _S, _DIM)


# speedup vs baseline: 1.4126x; 1.0035x over previous
import math

import jax
import jax.numpy as jnp
from jax.experimental import pallas as pl
from jax.experimental.pallas import tpu as pltpu

_DIM = 2048
_H = 16
_DH = 128
_S = 2048
_KEEP = max(1, int(_S * (1.0 - 0.6)))
_BLK = 256
_NB = _S // _BLK
_SCALE = 1.0 / math.sqrt(_DH)
_NEG = -1e9


def _key_to_f32(cand_u):
    return jax.lax.bitcast_convert_type(
        jnp.where(cand_u >= jnp.uint32(0x80000000),
                  cand_u & jnp.uint32(0x7FFFFFFF), ~cand_u),
        jnp.float32)


def _mono_kernel(x_ref, wq_ref, bq_ref, wk_ref, bk_ref, wv_ref, bv_ref,
                 wo_ref, bo_ref, o_ref, k_s, v_s, o_s):
    xh = x_ref[...]
    k_s[...] = jnp.dot(xh, wk_ref[0], preferred_element_type=jnp.float32) + bk_ref[0]
    v_s[...] = jnp.dot(xh, wv_ref[0], preferred_element_type=jnp.float32) + bv_ref[0]

    def block_scores(i):
        ncol = (i + 1) * _BLK
        q = jnp.dot(xh[i * _BLK:(i + 1) * _BLK, :], wq_ref[0],
                    preferred_element_type=jnp.float32) + bq_ref[0]
        scores = jax.lax.dot_general(
            q, k_s[0:ncol, :], (((1,), (1,)), ((), ())),
            preferred_element_type=jnp.float32) * _SCALE
        rows = i * _BLK + jax.lax.broadcasted_iota(jnp.int32, (_BLK, ncol), 0)
        cols = jax.lax.broadcasted_iota(jnp.int32, (_BLK, ncol), 1)
        scores = jnp.where(cols <= rows, scores, _NEG)
        return scores, jnp.max(scores, axis=1, keepdims=True)

    def block_finish(i, p):
        ncol = (i + 1) * _BLK
        denom = jnp.sum(p, axis=1, keepdims=True)
        o_s[i * _BLK:(i + 1) * _BLK, :] = jnp.dot(
            p, v_s[0:ncol, :], preferred_element_type=jnp.float32) / denom

    for i in range(3):
        scores, m = block_scores(i)
        block_finish(i, jnp.exp(scores - m))

    for pair in ((3, 4), (5, 6), (7,)):
        sc = {i: block_scores(i) for i in pair}
        prefix = {i: jnp.zeros((_BLK, 1), jnp.uint32) for i in pair}
        for bit in range(31, -1, -1):
            for i in pair:
                fc = _key_to_f32(prefix[i] | jnp.uint32(1 << bit))
                cnt = jnp.sum((sc[i][0] >= fc).astype(jnp.float32),
                              axis=1, keepdims=True)
                prefix[i] = jnp.where(cnt >= float(_KEEP),
                                      prefix[i] | jnp.uint32(1 << bit),
                                      prefix[i])
        for i in pair:
            scores, m = sc[i]
            p = jnp.where(scores >= _key_to_f32(prefix[i]),
                          jnp.exp(scores - m), 0.0)
            block_finish(i, p)

    o_ref[...] = jax.lax.dot_general(
        o_s[...], wo_ref[...], (((0,), (1,)), ((), ())),
        preferred_element_type=jnp.float32) + bo_ref[0]


def kernel(x, causal_mask, Wq, bq, Wk, bk, Wv, bv, Wo, bo):
    x2 = x.reshape(_S, _DIM)
    w_spec = pl.BlockSpec((1, _DH, _DH), lambda h: (h, 0, 0))
    b_spec = pl.BlockSpec((1, 1, _DH), lambda h: (h, 0, 0))
    final = pl.pallas_call(
        _mono_kernel,
        grid=(_H,),
        in_specs=[
            pl.BlockSpec((_S, _DH), lambda h: (0, h)),
            w_spec, b_spec, w_spec, b_spec, w_spec, b_spec,
            pl.BlockSpec((_DIM, _DIM), lambda h: (0, 0)),
            pl.BlockSpec((1, _DIM), lambda h: (0, 0)),
        ],
        out_specs=pl.BlockSpec((_DH, _DIM), lambda h: (h, 0)),
        out_shape=jax.ShapeDtypeStruct((_S, _DIM), jnp.float32),
        scratch_shapes=[
            pltpu.VMEM((_S, _DH), jnp.float32),
            pltpu.VMEM((_S, _DH), jnp.float32),
            pltpu.VMEM((_S, _DH), jnp.float32),
        ],
    )(x2, Wq, bq.reshape(_H, 1, _DH), Wk, bk.reshape(_H, 1, _DH),
      Wv, bv.reshape(_H, 1, _DH), Wo, bo.reshape(1, _DIM))
    return final.reshape(1, _S, _DIM)
